# baseline (speedup 1.0000x reference)
import functools

import jax
import jax.numpy as jnp
from jax import lax
from jax.experimental import pallas as pl
from jax.experimental.pallas import tpu as pltpu
from jax.experimental.pallas import tpu_sc as plsc

_N = 10000
_D = 128
_NP = 10240
_DUMMY = _N
_SHIFT = 14
_MASKV = (1 << _SHIFT) - 1
_DUPBIT = 1 << 30

_NC = 2
_NS = 16
_NW = _NC * _NS
_CK = 64
_RB = 2 * _CK
_NB = 2


def _sc_aggregate(keys_per_tile):
    nchunks = keys_per_tile // _CK
    assert nchunks % _NB == 0 and nchunks // _NB >= 2
    rows_per_tile = _NP // _NS
    mesh = plsc.VectorSubcoreMesh(core_axis_name="c", subcore_axis_name="s")

    @functools.partial(
        pl.kernel,
        out_type=jax.ShapeDtypeStruct((_NC * _NP, _D), jnp.float32),
        mesh=mesh,
        scratch_types=[
            pltpu.VMEM((keys_per_tile,), jnp.int32),
            pltpu.VMEM((_NB, _RB), jnp.int32),
            pltpu.VMEM((_NB, _RB), jnp.int32),
            [pltpu.VMEM((_RB, _D), jnp.float32) for _ in range(_NB)],
            pltpu.VMEM_SHARED((_NP, _D), jnp.float32),
            [pltpu.SemaphoreType.DMA for _ in range(_NB)],
        ],
    )
    def agg(table_hbm, skey_hbm, out_hbm, kbuf, sidx, didx, bufs, acc, sems):
        cid = lax.axis_index("c")
        sid = lax.axis_index("s")
        wid = cid * _NS + sid
        base = wid * keys_per_tile
        pltpu.sync_copy(skey_hbm.at[pl.ds(base, keys_per_tile)], kbuf)

        dummy = jnp.full((16,), _DUMMY, jnp.int32)
        zero = jnp.zeros((16,), jnp.int32)

        def decode(c, slot):
            for v in range(_CK // 16):
                k = kbuf[pl.ds(c * _CK + v * 16, 16)]
                dup = lax.bitwise_and(k, _DUPBIT) != zero
                a = lax.bitwise_and(lax.shift_right_logical(k, _SHIFT), _MASKV)
                b = lax.bitwise_and(k, _MASKV)
                sidx[slot, pl.ds(v * 16, 16)] = jnp.where(dup, dummy, b)
                sidx[slot, pl.ds(_CK + v * 16, 16)] = jnp.where(
                    jnp.logical_or(dup, a == b), dummy, a)
                didx[slot, pl.ds(v * 16, 16)] = a
                didx[slot, pl.ds(_CK + v * 16, 16)] = b

        def _zero(t, _):
            i = t // (_D // 16)
            j = t - i * (_D // 16)
            bufs[0][i, pl.ds(j * 16, 16)] = jnp.zeros((16,), jnp.float32)
            return 0
        lax.fori_loop(0, _RB * (_D // 16), _zero, 0)
        for r in range(rows_per_tile // _RB):
            pltpu.sync_copy(bufs[0], acc.at[pl.ds(sid * rows_per_tile + r * _RB, _RB)])
        plsc.subcore_barrier()

        for b in range(_NB):
            decode(b, b)
            pltpu.async_copy(table_hbm.at[sidx.at[b]], bufs[b], sems[b])

        def round_body(i, _):
            g0 = i * _NB
            for b in range(_NB):
                pltpu.make_async_copy(table_hbm.at[pl.ds(0, _RB)], bufs[b],
                                      sems[b]).wait()
                pltpu.sync_copy(bufs[b], acc.at[didx.at[b]], add=True)
                decode(g0 + b + _NB, b)
                pltpu.async_copy(table_hbm.at[sidx.at[b]], bufs[b], sems[b])
            return 0
        lax.fori_loop(0, nchunks // _NB - 1, round_body, 0)
        for b in range(_NB):
            pltpu.make_async_copy(table_hbm.at[pl.ds(0, _RB)], bufs[b],
                                  sems[b]).wait()
            pltpu.sync_copy(bufs[b], acc.at[didx.at[b]], add=True)

        plsc.subcore_barrier()
        pltpu.sync_copy(
            acc.at[pl.ds(sid * rows_per_tile, rows_per_tile)],
            out_hbm.at[pl.ds(cid * _NP + sid * rows_per_tile, rows_per_tile)])

    return agg


_BR = 256
_G = _NP // _BR


def _dot(a, w_ref):
    return jnp.dot(a.astype(jnp.bfloat16), w_ref[...],
                   preferred_element_type=jnp.float32)


def _tc_layer0(xp, parts, W1, b1, W2, b2, bns, bnb, WihT, bih, bhh):
    def body(x_ref, p0_ref, p1_ref, w1_ref, b1_ref, w2_ref, b2_ref,
             bns_ref, bnb_ref, wih_ref, bih_ref, bhh_ref, o_ref):
        t = x_ref[...] + p0_ref[...] + p1_ref[...]
        m = jnp.maximum(_dot(t, w1_ref) + b1_ref[...], 0.0)
        y = (_dot(m, w2_ref) + b2_ref[...]) * bns_ref[...] + bnb_ref[...]
        gi = _dot(y, wih_ref) + bih_ref[...]
        r = jax.nn.sigmoid(gi[:, 0:_D] + bhh_ref[:, 0:_D])
        z = jax.nn.sigmoid(gi[:, _D:2 * _D] + bhh_ref[:, _D:2 * _D])
        n = jnp.tanh(gi[:, 2 * _D:3 * _D] + r * bhh_ref[:, 2 * _D:3 * _D])
        h = (1.0 - z) * n
        rowid = pl.program_id(0) * _BR + lax.broadcasted_iota(jnp.int32, (_BR, _D), 0)
        o_ref[...] = jnp.where(rowid < _N, h, 0.0)

    full = lambda shape: pl.BlockSpec(shape, lambda i: (0, 0))
    return pl.pallas_call(
        body,
        grid=(_G,),
        in_specs=[
            pl.BlockSpec((_BR, _D), lambda i: (i, 0)),
            pl.BlockSpec((_BR, _D), lambda i: (i, 0)),
            pl.BlockSpec((_BR, _D), lambda i: (i + _G, 0)),
            full((_D, _D)), full((1, _D)), full((_D, _D)), full((1, _D)),
            full((1, _D)), full((1, _D)),
            full((_D, 3 * _D)), full((1, 3 * _D)), full((1, 3 * _D)),
        ],
        out_specs=pl.BlockSpec((_BR, _D), lambda i: (i, 0)),
        out_shape=jax.ShapeDtypeStruct((_NP, _D), jnp.float32),
    )(xp, parts, parts, W1, b1, W2, b2, bns, bnb, WihT, bih, bhh)


def _tc_layer1(hp, parts, W1, b1, W2, b2, bns, bnb, WihT, bih, WhhT, bhh,
               lW1, lb1, lW2, lb2):
    def body(h_ref, p0_ref, p1_ref, w1_ref, b1_ref, w2_ref, b2_ref,
             bns_ref, bnb_ref, wih_ref, bih_ref, whh_ref, bhh_ref,
             lw1_ref, lb1_ref, lw2_ref, lb2_ref, o_ref):
        h = h_ref[...]
        t = h + p0_ref[...] + p1_ref[...]
        m = jnp.maximum(_dot(t, w1_ref) + b1_ref[...], 0.0)
        y = (_dot(m, w2_ref) + b2_ref[...]) * bns_ref[...] + bnb_ref[...]
        gi = _dot(y, wih_ref) + bih_ref[...]
        gh = _dot(h, whh_ref) + bhh_ref[...]
        r = jax.nn.sigmoid(gi[:, 0:_D] + gh[:, 0:_D])
        z = jax.nn.sigmoid(gi[:, _D:2 * _D] + gh[:, _D:2 * _D])
        n = jnp.tanh(gi[:, 2 * _D:3 * _D] + r * gh[:, 2 * _D:3 * _D])
        h2 = (1.0 - z) * n + z * h
        v = jnp.maximum(_dot(h2, lw1_ref) + lb1_ref[...], 0.0)
        o_ref[...] = _dot(v, lw2_ref) + lb2_ref[...]

    full = lambda shape: pl.BlockSpec(shape, lambda i: (0, 0))
    return pl.pallas_call(
        body,
        grid=(_G,),
        in_specs=[
            pl.BlockSpec((_BR, _D), lambda i: (i, 0)),
            pl.BlockSpec((_BR, _D), lambda i: (i, 0)),
            pl.BlockSpec((_BR, _D), lambda i: (i + _G, 0)),
            full((_D, _D)), full((1, _D)), full((_D, _D)), full((1, _D)),
            full((1, _D)), full((1, _D)),
            full((_D, 3 * _D)), full((1, 3 * _D)),
            full((_D, 3 * _D)), full((1, 3 * _D)),
            full((_D, _D)), full((1, _D)), full((_D, _D)), full((1, _D)),
        ],
        out_specs=pl.BlockSpec((_BR, _D), lambda i: (i, 0)),
        out_shape=jax.ShapeDtypeStruct((_NP, _D), jnp.float32),
    )(hp, parts, parts, W1, b1, W2, b2, bns, bnb, WihT, bih, WhhT, bhh,
      lW1, lb1, lW2, lb2)


def kernel(x, edge_index, mlp0_W1, mlp0_b1, mlp0_W2, mlp0_b2, bn0_gamma, bn0_beta,
           mlp1_W1, mlp1_b1, mlp1_W2, mlp1_b2, bn1_gamma, bn1_beta,
           gru_W_ih, gru_W_hh, gru_b_ih, gru_b_hh,
           last_W1, last_b1, last_W2, last_b2):
    E = edge_index.shape[1]

    s = edge_index[0]
    d = edge_index[1]
    ckey = jnp.bitwise_or(
        jnp.left_shift(jnp.minimum(s, d), _SHIFT), jnp.maximum(s, d))
    skey = jnp.sort(ckey)
    dupf = jnp.concatenate([jnp.zeros((1,), jnp.int32),
                            (skey[1:] == skey[:-1]).astype(jnp.int32)])
    skey = jnp.bitwise_or(skey, dupf * _DUPBIT)

    keys_per_tile = -(-E // _NW)
    keys_per_tile = -(-keys_per_tile // (_CK * _NB)) * (_CK * _NB)
    total = _NW * keys_per_tile
    skeyp = jnp.concatenate([
        skey,
        jnp.broadcast_to(jnp.bitwise_or(skey[-1], _DUPBIT), (total - E,)),
    ])

    xp = jnp.concatenate([x, jnp.zeros((_NP - _N, _D), jnp.float32)], axis=0)
    xr = lax.optimization_barrier(xp.astype(jnp.bfloat16)).astype(jnp.float32)

    bf = lambda w: w.astype(jnp.bfloat16)
    bns0 = (bn0_gamma * (1.0 / jnp.sqrt(1.0 + 1e-5)))[None, :]
    bnb0 = bn0_beta[None, :]
    bns1 = (bn1_gamma * (1.0 / jnp.sqrt(1.0 + 1e-5)))[None, :]
    bnb1 = bn1_beta[None, :]
    WihT = bf(gru_W_ih.T)
    WhhT = bf(gru_W_hh.T)
    bih = gru_b_ih[None, :]
    bhh = gru_b_hh[None, :]
    b1_0 = mlp0_b1[None, :]
    b2_0 = mlp0_b2[None, :]
    b1_1 = mlp1_b1[None, :]
    b2_1 = mlp1_b2[None, :]
    lb1 = last_b1[None, :]
    lb2 = last_b2[None, :]

    agg = _sc_aggregate(keys_per_tile)

    parts0 = agg(xr, skeyp)
    hp = _tc_layer0(xp, parts0, bf(mlp0_W1), b1_0, bf(mlp0_W2), b2_0,
                    bns0, bnb0, WihT, bih, bhh)
    hr = lax.optimization_barrier(hp.astype(jnp.bfloat16)).astype(jnp.float32)
    parts1 = agg(hr, skeyp)
    outp = _tc_layer1(hp, parts1, bf(mlp1_W1), b1_1, bf(mlp1_W2), b2_1,
                      bns1, bnb1, WihT, bih, WhhT, bhh,
                      bf(last_W1), lb1, bf(last_W2), lb2)
    return outp[:_N]


# speedup vs baseline: 1.0000x; 1.0000x over previous
import functools

import jax
import jax.numpy as jnp
from jax import lax
from jax.experimental import pallas as pl
from jax.experimental.pallas import tpu as pltpu
from jax.experimental.pallas import tpu_sc as plsc

_N = 10000
_D = 128
_NP = 10240
_DUMMY = _N
_SHIFT = 14
_MASKV = (1 << _SHIFT) - 1
_DUPBIT = 1 << 30

_NC = 2
_NS = 16
_NW = _NC * _NS
_CK = 64
_RB = 2 * _CK
_NB = 2


def _sc_aggregate(keys_per_tile):
    nchunks = keys_per_tile // _CK
    assert nchunks % _NB == 0 and nchunks // _NB >= 2
    rows_per_tile = _NP // _NS
    mesh = plsc.VectorSubcoreMesh(core_axis_name="c", subcore_axis_name="s")

    @functools.partial(
        pl.kernel,
        out_type=jax.ShapeDtypeStruct((_NC * _NP, _D), jnp.float32),
        mesh=mesh,
        scratch_types=[
            pltpu.VMEM((keys_per_tile,), jnp.int32),
            pltpu.VMEM((_NB, _RB), jnp.int32),
            pltpu.VMEM((_NB, _RB), jnp.int32),
            [pltpu.VMEM((_RB, _D), jnp.float32) for _ in range(_NB)],
            pltpu.VMEM_SHARED((_NP, _D), jnp.float32),
            [pltpu.SemaphoreType.DMA for _ in range(_NB)],
        ],
    )
    def agg(table_hbm, skey_hbm, out_hbm, kbuf, sidx, didx, bufs, acc, sems):
        cid = lax.axis_index("c")
        sid = lax.axis_index("s")
        wid = sid * _NC + cid
        base = wid * keys_per_tile
        pltpu.sync_copy(skey_hbm.at[pl.ds(base, keys_per_tile)], kbuf)

        dummy = jnp.full((16,), _DUMMY, jnp.int32)
        zero = jnp.zeros((16,), jnp.int32)

        def decode(c, slot):
            for v in range(_CK // 16):
                k = kbuf[pl.ds(c * _CK + v * 16, 16)]
                dup = lax.bitwise_and(k, _DUPBIT) != zero
                a = lax.bitwise_and(lax.shift_right_logical(k, _SHIFT), _MASKV)
                b = lax.bitwise_and(k, _MASKV)
                sidx[slot, pl.ds(v * 16, 16)] = jnp.where(dup, dummy, b)
                sidx[slot, pl.ds(_CK + v * 16, 16)] = jnp.where(
                    jnp.logical_or(dup, a == b), dummy, a)
                didx[slot, pl.ds(v * 16, 16)] = a
                didx[slot, pl.ds(_CK + v * 16, 16)] = b

        def _zero(t, _):
            i = t // (_D // 16)
            j = t - i * (_D // 16)
            bufs[0][i, pl.ds(j * 16, 16)] = jnp.zeros((16,), jnp.float32)
            return 0
        lax.fori_loop(0, _RB * (_D // 16), _zero, 0)
        for r in range(rows_per_tile // _RB):
            pltpu.sync_copy(bufs[0], acc.at[pl.ds(sid * rows_per_tile + r * _RB, _RB)])
        plsc.subcore_barrier()

        for b in range(_NB):
            decode(b, b)
            pltpu.async_copy(table_hbm.at[sidx.at[b]], bufs[b], sems[b])

        def round_body(i, _):
            g0 = i * _NB
            for b in range(_NB):
                pltpu.make_async_copy(table_hbm.at[pl.ds(0, _RB)], bufs[b],
                                      sems[b]).wait()
                pltpu.sync_copy(bufs[b], acc.at[didx.at[b]], add=True)
                decode(g0 + b + _NB, b)
                pltpu.async_copy(table_hbm.at[sidx.at[b]], bufs[b], sems[b])
            return 0
        lax.fori_loop(0, nchunks // _NB - 1, round_body, 0)
        for b in range(_NB):
            pltpu.make_async_copy(table_hbm.at[pl.ds(0, _RB)], bufs[b],
                                  sems[b]).wait()
            pltpu.sync_copy(bufs[b], acc.at[didx.at[b]], add=True)

        plsc.subcore_barrier()
        pltpu.sync_copy(
            acc.at[pl.ds(sid * rows_per_tile, rows_per_tile)],
            out_hbm.at[pl.ds(cid * _NP + sid * rows_per_tile, rows_per_tile)])

    return agg


_BR = 256
_G = _NP // _BR


def _dot(a, w_ref):
    return jnp.dot(a.astype(jnp.bfloat16), w_ref[...],
                   preferred_element_type=jnp.float32)


def _tc_layer0(xp, parts, W1, b1, W2, b2, bns, bnb, WihT, bih, bhh):
    def body(x_ref, p0_ref, p1_ref, w1_ref, b1_ref, w2_ref, b2_ref,
             bns_ref, bnb_ref, wih_ref, bih_ref, bhh_ref, o_ref):
        t = x_ref[...] + p0_ref[...] + p1_ref[...]
        m = jnp.maximum(_dot(t, w1_ref) + b1_ref[...], 0.0)
        y = (_dot(m, w2_ref) + b2_ref[...]) * bns_ref[...] + bnb_ref[...]
        gi = _dot(y, wih_ref) + bih_ref[...]
        r = jax.nn.sigmoid(gi[:, 0:_D] + bhh_ref[:, 0:_D])
        z = jax.nn.sigmoid(gi[:, _D:2 * _D] + bhh_ref[:, _D:2 * _D])
        n = jnp.tanh(gi[:, 2 * _D:3 * _D] + r * bhh_ref[:, 2 * _D:3 * _D])
        h = (1.0 - z) * n
        rowid = pl.program_id(0) * _BR + lax.broadcasted_iota(jnp.int32, (_BR, _D), 0)
        o_ref[...] = jnp.where(rowid < _N, h, 0.0)

    full = lambda shape: pl.BlockSpec(shape, lambda i: (0, 0))
    return pl.pallas_call(
        body,
        grid=(_G,),
        in_specs=[
            pl.BlockSpec((_BR, _D), lambda i: (i, 0)),
            pl.BlockSpec((_BR, _D), lambda i: (i, 0)),
            pl.BlockSpec((_BR, _D), lambda i: (i + _G, 0)),
            full((_D, _D)), full((1, _D)), full((_D, _D)), full((1, _D)),
            full((1, _D)), full((1, _D)),
            full((_D, 3 * _D)), full((1, 3 * _D)), full((1, 3 * _D)),
        ],
        out_specs=pl.BlockSpec((_BR, _D), lambda i: (i, 0)),
        out_shape=jax.ShapeDtypeStruct((_NP, _D), jnp.float32),
    )(xp, parts, parts, W1, b1, W2, b2, bns, bnb, WihT, bih, bhh)


def _tc_layer1(hp, parts, W1, b1, W2, b2, bns, bnb, WihT, bih, WhhT, bhh,
               lW1, lb1, lW2, lb2):
    def body(h_ref, p0_ref, p1_ref, w1_ref, b1_ref, w2_ref, b2_ref,
             bns_ref, bnb_ref, wih_ref, bih_ref, whh_ref, bhh_ref,
             lw1_ref, lb1_ref, lw2_ref, lb2_ref, o_ref):
        h = h_ref[...]
        t = h + p0_ref[...] + p1_ref[...]
        m = jnp.maximum(_dot(t, w1_ref) + b1_ref[...], 0.0)
        y = (_dot(m, w2_ref) + b2_ref[...]) * bns_ref[...] + bnb_ref[...]
        gi = _dot(y, wih_ref) + bih_ref[...]
        gh = _dot(h, whh_ref) + bhh_ref[...]
        r = jax.nn.sigmoid(gi[:, 0:_D] + gh[:, 0:_D])
        z = jax.nn.sigmoid(gi[:, _D:2 * _D] + gh[:, _D:2 * _D])
        n = jnp.tanh(gi[:, 2 * _D:3 * _D] + r * gh[:, 2 * _D:3 * _D])
        h2 = (1.0 - z) * n + z * h
        v = jnp.maximum(_dot(h2, lw1_ref) + lb1_ref[...], 0.0)
        o_ref[...] = _dot(v, lw2_ref) + lb2_ref[...]

    full = lambda shape: pl.BlockSpec(shape, lambda i: (0, 0))
    return pl.pallas_call(
        body,
        grid=(_G,),
        in_specs=[
            pl.BlockSpec((_BR, _D), lambda i: (i, 0)),
            pl.BlockSpec((_BR, _D), lambda i: (i, 0)),
            pl.BlockSpec((_BR, _D), lambda i: (i + _G, 0)),
            full((_D, _D)), full((1, _D)), full((_D, _D)), full((1, _D)),
            full((1, _D)), full((1, _D)),
            full((_D, 3 * _D)), full((1, 3 * _D)),
            full((_D, 3 * _D)), full((1, 3 * _D)),
            full((_D, _D)), full((1, _D)), full((_D, _D)), full((1, _D)),
        ],
        out_specs=pl.BlockSpec((_BR, _D), lambda i: (i, 0)),
        out_shape=jax.ShapeDtypeStruct((_NP, _D), jnp.float32),
    )(hp, parts, parts, W1, b1, W2, b2, bns, bnb, WihT, bih, WhhT, bhh,
      lW1, lb1, lW2, lb2)


def kernel(x, edge_index, mlp0_W1, mlp0_b1, mlp0_W2, mlp0_b2, bn0_gamma, bn0_beta,
           mlp1_W1, mlp1_b1, mlp1_W2, mlp1_b2, bn1_gamma, bn1_beta,
           gru_W_ih, gru_W_hh, gru_b_ih, gru_b_hh,
           last_W1, last_b1, last_W2, last_b2):
    E = edge_index.shape[1]

    s = edge_index[0]
    d = edge_index[1]
    ckey = jnp.bitwise_or(
        jnp.left_shift(jnp.minimum(s, d), _SHIFT), jnp.maximum(s, d))
    skey = jnp.sort(ckey)
    dupf = jnp.concatenate([jnp.zeros((1,), jnp.int32),
                            (skey[1:] == skey[:-1]).astype(jnp.int32)])
    skey = jnp.bitwise_or(skey, dupf * _DUPBIT)

    keys_per_tile = -(-E // _NW)
    keys_per_tile = -(-keys_per_tile // (_CK * _NB)) * (_CK * _NB)
    total = _NW * keys_per_tile
    skeyp = jnp.concatenate([
        skey,
        jnp.broadcast_to(jnp.bitwise_or(skey[-1], _DUPBIT), (total - E,)),
    ])

    xp = jnp.concatenate([x, jnp.zeros((_NP - _N, _D), jnp.float32)], axis=0)
    xr = lax.optimization_barrier(xp.astype(jnp.bfloat16)).astype(jnp.float32)

    bf = lambda w: w.astype(jnp.bfloat16)
    bns0 = (bn0_gamma * (1.0 / jnp.sqrt(1.0 + 1e-5)))[None, :]
    bnb0 = bn0_beta[None, :]
    bns1 = (bn1_gamma * (1.0 / jnp.sqrt(1.0 + 1e-5)))[None, :]
    bnb1 = bn1_beta[None, :]
    WihT = bf(gru_W_ih.T)
    WhhT = bf(gru_W_hh.T)
    bih = gru_b_ih[None, :]
    bhh = gru_b_hh[None, :]
    b1_0 = mlp0_b1[None, :]
    b2_0 = mlp0_b2[None, :]
    b1_1 = mlp1_b1[None, :]
    b2_1 = mlp1_b2[None, :]
    lb1 = last_b1[None, :]
    lb2 = last_b2[None, :]

    agg = _sc_aggregate(keys_per_tile)

    parts0 = agg(xr, skeyp)
    hp = _tc_layer0(xp, parts0, bf(mlp0_W1), b1_0, bf(mlp0_W2), b2_0,
                    bns0, bnb0, WihT, bih, bhh)
    hr = lax.optimization_barrier(hp.astype(jnp.bfloat16)).astype(jnp.float32)
    parts1 = agg(hr, skeyp)
    outp = _tc_layer1(hp, parts1, bf(mlp1_W1), b1_1, bf(mlp1_W2), b2_1,
                      bns1, bnb1, WihT, bih, WhhT, bhh,
                      bf(last_W1), lb1, bf(last_W2), lb2)
    return outp[:_N]


# speedup vs baseline: 1.2226x; 1.2226x over previous
import functools

import jax
import jax.numpy as jnp
from jax import lax
from jax.experimental import pallas as pl
from jax.experimental.pallas import tpu as pltpu
from jax.experimental.pallas import tpu_sc as plsc

_N = 10000
_D = 128
_NP = 10240
_DUMMY = _N
_SHIFT = 14
_MASKV = (1 << _SHIFT) - 1
_DUPBIT = 1 << 30

_NC = 2
_NS = 16
_NW = _NC * _NS
_CK = 64
_RB = 2 * _CK


def _sc_aggregate(keys_per_tile):
    nchunks = keys_per_tile // _CK
    rows_per_tile = _NP // _NS
    mesh = plsc.VectorSubcoreMesh(core_axis_name="c", subcore_axis_name="s")

    @functools.partial(
        pl.kernel,
        out_type=jax.ShapeDtypeStruct((_NC * _NP, _D), jnp.float32),
        mesh=mesh,
        scratch_types=[
            pltpu.VMEM((keys_per_tile,), jnp.int32),
            pltpu.VMEM((_RB,), jnp.int32),
            pltpu.VMEM((_RB,), jnp.int32),
            pltpu.VMEM((_RB, _D), jnp.float32),
            pltpu.VMEM_SHARED((_NP, _D), jnp.float32),
            pltpu.SemaphoreType.DMA,
        ],
    )
    def agg(table_hbm, skey_hbm, out_hbm, kbuf, sidx, didx, rows, acc, sem):
        cid = lax.axis_index("c")
        sid = lax.axis_index("s")
        wid = cid * _NS + sid
        base = wid * keys_per_tile
        pltpu.sync_copy(skey_hbm.at[pl.ds(base, keys_per_tile)], kbuf)

        def _zero(t, _):
            i = t // (_D // 16)
            j = t - i * (_D // 16)
            rows[i, pl.ds(j * 16, 16)] = jnp.zeros((16,), jnp.float32)
            return 0
        lax.fori_loop(0, _RB * (_D // 16), _zero, 0)
        for r in range(rows_per_tile // _RB):
            pltpu.sync_copy(rows, acc.at[pl.ds(sid * rows_per_tile + r * _RB, _RB)])
        plsc.subcore_barrier()

        dummy = jnp.full((16,), _DUMMY, jnp.int32)
        zero = jnp.zeros((16,), jnp.int32)

        def chunk(c, _):
            for v in range(_CK // 16):
                off = c * _CK + v * 16
                k = kbuf[pl.ds(off, 16)]
                dup = lax.bitwise_and(k, _DUPBIT) != zero
                a = lax.bitwise_and(lax.shift_right_logical(k, _SHIFT), _MASKV)
                b = lax.bitwise_and(k, _MASKV)
                src0 = jnp.where(dup, dummy, b)
                src1 = jnp.where(jnp.logical_or(dup, a == b), dummy, a)
                sidx[pl.ds(v * 16, 16)] = src0
                sidx[pl.ds(_CK + v * 16, 16)] = src1
                didx[pl.ds(v * 16, 16)] = a
                didx[pl.ds(_CK + v * 16, 16)] = b
            pltpu.async_copy(table_hbm.at[sidx], rows, sem).wait()
            pltpu.sync_copy(rows, acc.at[didx], add=True)
            return 0
        lax.fori_loop(0, nchunks, chunk, 0)

        plsc.subcore_barrier()
        pltpu.sync_copy(
            acc.at[pl.ds(sid * rows_per_tile, rows_per_tile)],
            out_hbm.at[pl.ds(cid * _NP + sid * rows_per_tile, rows_per_tile)])

    return agg


_BR = 256
_G = _NP // _BR


def _dot(a, w_ref):
    return jnp.dot(a.astype(jnp.bfloat16), w_ref[...],
                   preferred_element_type=jnp.float32)


def _tc_layer0(xp, parts, W1, b1, W2, b2, bns, bnb, WihT, bih, bhh):
    def body(x_ref, p0_ref, p1_ref, w1_ref, b1_ref, w2_ref, b2_ref,
             bns_ref, bnb_ref, wih_ref, bih_ref, bhh_ref, o_ref):
        t = x_ref[...] + p0_ref[...] + p1_ref[...]
        m = jnp.maximum(_dot(t, w1_ref) + b1_ref[...], 0.0)
        y = (_dot(m, w2_ref) + b2_ref[...]) * bns_ref[...] + bnb_ref[...]
        gi = _dot(y, wih_ref) + bih_ref[...]
        r = jax.nn.sigmoid(gi[:, 0:_D] + bhh_ref[:, 0:_D])
        z = jax.nn.sigmoid(gi[:, _D:2 * _D] + bhh_ref[:, _D:2 * _D])
        n = jnp.tanh(gi[:, 2 * _D:3 * _D] + r * bhh_ref[:, 2 * _D:3 * _D])
        h = (1.0 - z) * n
        rowid = pl.program_id(0) * _BR + lax.broadcasted_iota(jnp.int32, (_BR, _D), 0)
        o_ref[...] = jnp.where(rowid < _N, h, 0.0)

    full = lambda shape: pl.BlockSpec(shape, lambda i: (0, 0))
    return pl.pallas_call(
        body,
        grid=(_G,),
        in_specs=[
            pl.BlockSpec((_BR, _D), lambda i: (i, 0)),
            pl.BlockSpec((_BR, _D), lambda i: (i, 0)),
            pl.BlockSpec((_BR, _D), lambda i: (i + _G, 0)),
            full((_D, _D)), full((1, _D)), full((_D, _D)), full((1, _D)),
            full((1, _D)), full((1, _D)),
            full((_D, 3 * _D)), full((1, 3 * _D)), full((1, 3 * _D)),
        ],
        out_specs=pl.BlockSpec((_BR, _D), lambda i: (i, 0)),
        out_shape=jax.ShapeDtypeStruct((_NP, _D), jnp.float32),
    )(xp, parts, parts, W1, b1, W2, b2, bns, bnb, WihT, bih, bhh)


def _tc_layer1(hp, parts, W1, b1, W2, b2, bns, bnb, WihT, bih, WhhT, bhh,
               lW1, lb1, lW2, lb2):
    def body(h_ref, p0_ref, p1_ref, w1_ref, b1_ref, w2_ref, b2_ref,
             bns_ref, bnb_ref, wih_ref, bih_ref, whh_ref, bhh_ref,
             lw1_ref, lb1_ref, lw2_ref, lb2_ref, o_ref):
        h = h_ref[...]
        t = h + p0_ref[...] + p1_ref[...]
        m = jnp.maximum(_dot(t, w1_ref) + b1_ref[...], 0.0)
        y = (_dot(m, w2_ref) + b2_ref[...]) * bns_ref[...] + bnb_ref[...]
        gi = _dot(y, wih_ref) + bih_ref[...]
        gh = _dot(h, whh_ref) + bhh_ref[...]
        r = jax.nn.sigmoid(gi[:, 0:_D] + gh[:, 0:_D])
        z = jax.nn.sigmoid(gi[:, _D:2 * _D] + gh[:, _D:2 * _D])
        n = jnp.tanh(gi[:, 2 * _D:3 * _D] + r * gh[:, 2 * _D:3 * _D])
        h2 = (1.0 - z) * n + z * h
        v = jnp.maximum(_dot(h2, lw1_ref) + lb1_ref[...], 0.0)
        o_ref[...] = _dot(v, lw2_ref) + lb2_ref[...]

    full = lambda shape: pl.BlockSpec(shape, lambda i: (0, 0))
    return pl.pallas_call(
        body,
        grid=(_G,),
        in_specs=[
            pl.BlockSpec((_BR, _D), lambda i: (i, 0)),
            pl.BlockSpec((_BR, _D), lambda i: (i, 0)),
            pl.BlockSpec((_BR, _D), lambda i: (i + _G, 0)),
            full((_D, _D)), full((1, _D)), full((_D, _D)), full((1, _D)),
            full((1, _D)), full((1, _D)),
            full((_D, 3 * _D)), full((1, 3 * _D)),
            full((_D, 3 * _D)), full((1, 3 * _D)),
            full((_D, _D)), full((1, _D)), full((_D, _D)), full((1, _D)),
        ],
        out_specs=pl.BlockSpec((_BR, _D), lambda i: (i, 0)),
        out_shape=jax.ShapeDtypeStruct((_NP, _D), jnp.float32),
    )(hp, parts, parts, W1, b1, W2, b2, bns, bnb, WihT, bih, WhhT, bhh,
      lW1, lb1, lW2, lb2)


def kernel(x, edge_index, mlp0_W1, mlp0_b1, mlp0_W2, mlp0_b2, bn0_gamma, bn0_beta,
           mlp1_W1, mlp1_b1, mlp1_W2, mlp1_b2, bn1_gamma, bn1_beta,
           gru_W_ih, gru_W_hh, gru_b_ih, gru_b_hh,
           last_W1, last_b1, last_W2, last_b2):
    E = edge_index.shape[1]

    s = edge_index[0]
    d = edge_index[1]
    ckey = jnp.bitwise_or(
        jnp.left_shift(jnp.minimum(s, d), _SHIFT), jnp.maximum(s, d))
    skey = jnp.sort(ckey)
    dupf = jnp.concatenate([jnp.zeros((1,), jnp.int32),
                            (skey[1:] == skey[:-1]).astype(jnp.int32)])
    skey = jnp.bitwise_or(skey, dupf * _DUPBIT)

    keys_per_tile = -(-E // _NW)
    keys_per_tile = -(-keys_per_tile // _CK) * _CK
    total = _NW * keys_per_tile
    skeyp = jnp.concatenate([
        skey,
        jnp.broadcast_to(jnp.bitwise_or(skey[-1], _DUPBIT), (total - E,)),
    ])

    xp = jnp.concatenate([x, jnp.zeros((_NP - _N, _D), jnp.float32)], axis=0)
    xr = lax.optimization_barrier(xp.astype(jnp.bfloat16)).astype(jnp.float32)

    bf = lambda w: w.astype(jnp.bfloat16)
    bns0 = (bn0_gamma * (1.0 / jnp.sqrt(1.0 + 1e-5)))[None, :]
    bnb0 = bn0_beta[None, :]
    bns1 = (bn1_gamma * (1.0 / jnp.sqrt(1.0 + 1e-5)))[None, :]
    bnb1 = bn1_beta[None, :]
    WihT = bf(gru_W_ih.T)
    WhhT = bf(gru_W_hh.T)
    bih = gru_b_ih[None, :]
    bhh = gru_b_hh[None, :]
    b1_0 = mlp0_b1[None, :]
    b2_0 = mlp0_b2[None, :]
    b1_1 = mlp1_b1[None, :]
    b2_1 = mlp1_b2[None, :]
    lb1 = last_b1[None, :]
    lb2 = last_b2[None, :]

    agg = _sc_aggregate(keys_per_tile)

    parts0 = agg(xr, skeyp)
    hp = _tc_layer0(xp, parts0, bf(mlp0_W1), b1_0, bf(mlp0_W2), b2_0,
                    bns0, bnb0, WihT, bih, bhh)
    hr = lax.optimization_barrier(hp.astype(jnp.bfloat16)).astype(jnp.float32)
    parts1 = agg(hr, skeyp)
    outp = _tc_layer1(hp, parts1, bf(mlp1_W1), b1_1, bf(mlp1_W2), b2_1,
                      bns1, bnb1, WihT, bih, WhhT, bhh,
                      bf(last_W1), lb1, bf(last_W2), lb2)
    return outp[:_N]
